# baseline (speedup 1.0000x reference)
import jax
import jax.numpy as jnp
from jax import lax
from jax.experimental import pallas as pl
from jax.experimental.pallas import tpu as pltpu
from jax.experimental.pallas import tpu_sc as plsc

NUM_CAMS = 100000
N_RAYS = 16384
L = 16
NC = 2
NS = 16
NW = NC * NS
BPW = N_RAYS // NW
IDXC = 128
NIDX = BPW // IDXC
NB = N_RAYS // 128
BPWB = BPW // 128

_A_COEF = (1.0 / 362880.0, -1.0 / 5040.0, 1.0 / 120.0, -1.0 / 6.0, 1.0)
_B_COEF = (1.0 / 3628800.0, -1.0 / 40320.0, 1.0 / 720.0, -1.0 / 24.0, 0.5)


def _poly(coef, t):
    acc = jnp.full((L,), coef[0], jnp.float32)
    for c in coef[1:]:
        acc = acc * t + c
    return acc


def _body(cam_hbm, rt_hbm, out_hbm, idx0, comp, ocomp, sems):
    wid = lax.axis_index("s") * NC + lax.axis_index("c")
    base = wid * BPW

    stages = [
        pltpu.async_copy(
            cam_hbm.at[pl.ds(base + k * IDXC, IDXC)],
            idx0.at[pl.ds(k * IDXC, IDXC)], sems.at[NIDX + k])
        for k in range(NIDX)
    ]

    copies = []
    for k in range(NIDX):
        stages[k].wait()
        irow = idx0.at[pl.ds(k * IDXC, IDXC)]
        sl = pl.ds(k * IDXC, IDXC)
        kc = []
        for j in range(6):
            kc.append(pltpu.async_copy(
                rt_hbm.at[j].at[irow], comp.at[j, sl], sems.at[k]))
        copies.append(kc)

    zero = jnp.zeros((L,), jnp.float32)
    one = jnp.full((L,), 1.0, jnp.float32)

    def chunk(b, ci, _):
        sl = pl.ds(b * IDXC + ci * L, L)
        r0 = comp[0, sl]
        r1 = comp[1, sl]
        r2 = comp[2, sl]
        t0 = comp[3, sl]
        t1 = comp[4, sl]
        t2 = comp[5, sl]

        th2 = r0 * r0 + r1 * r1 + r2 * r2
        A = _poly(_A_COEF, th2)
        B = _poly(_B_COEF, th2)

        ar0, ar1, ar2 = A * r0, A * r1, A * r2
        br0, br1, br2 = B * r0, B * r1, B * r2
        d = 1.0 - B * th2

        vals = (
            d + br0 * r0, br0 * r1 - ar2, br0 * r2 + ar1, t0,
            br1 * r0 + ar2, d + br1 * r1, br1 * r2 - ar0, t1,
            br2 * r0 - ar1, br2 * r1 + ar0, d + br2 * r2, t2,
            zero, zero, zero, one,
        )
        osl = pl.ds(ci * L, L)
        for q, v in enumerate(vals):
            ocomp[q // 4, b, q % 4, osl] = v
        return 0

    for k in range(NIDX):
        for c in copies[k]:
            c.wait()
        for ci in range(IDXC // L):
            chunk(k, ci, 0)

    pltpu.sync_copy(ocomp, out_hbm.at[:, pl.ds(wid * BPWB, BPWB)])


def kernel(cam_id, r, t):
    mesh = plsc.VectorSubcoreMesh(core_axis_name="c", subcore_axis_name="s")
    rt_flat = jnp.concatenate([jnp.transpose(r), jnp.transpose(t)], axis=0)
    out_t = pl.kernel(
        _body,
        out_type=jax.ShapeDtypeStruct((4, NB, 4, 128), jnp.float32),
        mesh=mesh,
        compiler_params=pltpu.CompilerParams(
            skip_device_barrier=True, use_tc_tiling_on_sc=False),
        scratch_types=[
            pltpu.VMEM((BPW,), jnp.int32),
            pltpu.VMEM((6, BPW), jnp.float32),
            pltpu.VMEM((4, BPWB, 4, 128), jnp.float32),
            pltpu.SemaphoreType.DMA((2 * NIDX,)),
        ],
    )(cam_id.astype(jnp.int32), rt_flat)
    return jnp.transpose(out_t, (1, 3, 0, 2)).reshape(N_RAYS, 4, 4)


# speedup vs baseline: 1.0047x; 1.0047x over previous
import jax
import jax.numpy as jnp
from jax import lax
from jax.experimental import pallas as pl
from jax.experimental.pallas import tpu as pltpu
from jax.experimental.pallas import tpu_sc as plsc

NUM_CAMS = 100000
N_RAYS = 16384
L = 16
NC = 2
NS = 16
NW = NC * NS
BPW = N_RAYS // NW
IDXC = 128
NIDX = BPW // IDXC
NB = N_RAYS // 128
BPWB = BPW // 128

_A_COEF = (1.0 / 362880.0, -1.0 / 5040.0, 1.0 / 120.0, -1.0 / 6.0, 1.0)
_B_COEF = (1.0 / 3628800.0, -1.0 / 40320.0, 1.0 / 720.0, -1.0 / 24.0, 0.5)


def _poly(coef, t):
    acc = jnp.full((L,), coef[0], jnp.float32)
    for c in coef[1:]:
        acc = acc * t + c
    return acc


def _body(cam_hbm, rt_hbm, out_hbm, idx0, comp, ocomp, sems):
    wid = lax.axis_index("s") * NC + lax.axis_index("c")
    base = wid * BPW

    pltpu.sync_copy(cam_hbm.at[pl.ds(base, BPW)], idx0)

    copies = []
    for k in range(NIDX):
        irow = idx0.at[pl.ds(k * IDXC, IDXC)]
        sl = pl.ds(k * IDXC, IDXC)
        kc = []
        for j in range(6):
            kc.append(pltpu.async_copy(
                rt_hbm.at[j].at[irow], comp.at[j, sl], sems.at[k]))
        copies.append(kc)

    zero = jnp.zeros((L,), jnp.float32)
    one = jnp.full((L,), 1.0, jnp.float32)

    def chunk(b, ci, _):
        sl = pl.ds(b * IDXC + ci * L, L)
        r0 = comp[0, sl]
        r1 = comp[1, sl]
        r2 = comp[2, sl]
        t0 = comp[3, sl]
        t1 = comp[4, sl]
        t2 = comp[5, sl]

        th2 = r0 * r0 + r1 * r1 + r2 * r2
        A = _poly(_A_COEF, th2)
        B = _poly(_B_COEF, th2)

        ar0, ar1, ar2 = A * r0, A * r1, A * r2
        br0, br1, br2 = B * r0, B * r1, B * r2
        d = 1.0 - B * th2

        vals = (
            d + br0 * r0, br0 * r1 - ar2, br0 * r2 + ar1, t0,
            br1 * r0 + ar2, d + br1 * r1, br1 * r2 - ar0, t1,
            br2 * r0 - ar1, br2 * r1 + ar0, d + br2 * r2, t2,
            zero, zero, zero, one,
        )
        osl = pl.ds(ci * L, L)
        for q, v in enumerate(vals):
            ocomp[q // 4, b, q % 4, osl] = v
        return 0

    for k in range(NIDX):
        for c in copies[k]:
            c.wait()
        lax.fori_loop(0, IDXC // L, lambda ci, a: chunk(k, ci, a), 0)

    pltpu.sync_copy(ocomp, out_hbm.at[:, pl.ds(wid * BPWB, BPWB)])


def kernel(cam_id, r, t):
    mesh = plsc.VectorSubcoreMesh(core_axis_name="c", subcore_axis_name="s")
    rt_flat = jnp.concatenate([jnp.transpose(r), jnp.transpose(t)], axis=0)
    out_t = pl.kernel(
        _body,
        out_type=jax.ShapeDtypeStruct((4, NB, 4, 128), jnp.float32),
        mesh=mesh,
        compiler_params=pltpu.CompilerParams(
            skip_device_barrier=True, use_tc_tiling_on_sc=False),
        scratch_types=[
            pltpu.VMEM((BPW,), jnp.int32),
            pltpu.VMEM((6, BPW), jnp.float32),
            pltpu.VMEM((4, BPWB, 4, 128), jnp.float32),
            pltpu.SemaphoreType.DMA((NIDX,)),
        ],
    )(cam_id.astype(jnp.int32), rt_flat)
    return jnp.transpose(out_t, (1, 3, 0, 2)).reshape(N_RAYS, 4, 4)


# speedup vs baseline: 1.0084x; 1.0037x over previous
import jax
import jax.numpy as jnp
from jax import lax
from jax.experimental import pallas as pl
from jax.experimental.pallas import tpu as pltpu
from jax.experimental.pallas import tpu_sc as plsc

NUM_CAMS = 100000
N_RAYS = 16384
L = 16
NC = 2
NS = 16
NW = NC * NS
BPW = N_RAYS // NW
IDXC = 128
NIDX = BPW // IDXC
NB = N_RAYS // 128
BPWB = BPW // 128

_A_COEF = (1.0 / 362880.0, -1.0 / 5040.0, 1.0 / 120.0, -1.0 / 6.0, 1.0)
_B_COEF = (1.0 / 3628800.0, -1.0 / 40320.0, 1.0 / 720.0, -1.0 / 24.0, 0.5)


def _poly(coef, t):
    acc = jnp.full((L,), coef[0], jnp.float32)
    for c in coef[1:]:
        acc = acc * t + c
    return acc


def _body(cam_hbm, rt_hbm, out_hbm, idx0, comp, ocomp, sems):
    wid = lax.axis_index("s") * NC + lax.axis_index("c")
    base = wid * BPW

    pltpu.sync_copy(cam_hbm.at[pl.ds(base, BPW)], idx0)

    copies = [
        pltpu.async_copy(rt_hbm.at[j].at[idx0], comp.at[j], sems.at[j])
        for j in range(6)
    ]

    zero = jnp.zeros((L,), jnp.float32)
    one = jnp.full((L,), 1.0, jnp.float32)

    def chunk(b, ci, _):
        sl = pl.ds(b * IDXC + ci * L, L)
        r0 = comp[0, sl]
        r1 = comp[1, sl]
        r2 = comp[2, sl]
        t0 = comp[3, sl]
        t1 = comp[4, sl]
        t2 = comp[5, sl]

        th2 = r0 * r0 + r1 * r1 + r2 * r2
        A = _poly(_A_COEF, th2)
        B = _poly(_B_COEF, th2)

        ar0, ar1, ar2 = A * r0, A * r1, A * r2
        br0, br1, br2 = B * r0, B * r1, B * r2
        d = 1.0 - B * th2

        vals = (
            d + br0 * r0, br0 * r1 - ar2, br0 * r2 + ar1, t0,
            br1 * r0 + ar2, d + br1 * r1, br1 * r2 - ar0, t1,
            br2 * r0 - ar1, br2 * r1 + ar0, d + br2 * r2, t2,
            zero, zero, zero, one,
        )
        osl = pl.ds(ci * L, L)
        for q, v in enumerate(vals):
            ocomp[q // 4, b, q % 4, osl] = v
        return 0

    for c in copies:
        c.wait()
    for k in range(NIDX):
        lax.fori_loop(0, IDXC // L, lambda ci, a: chunk(k, ci, a), 0)

    pltpu.sync_copy(ocomp, out_hbm.at[:, pl.ds(wid * BPWB, BPWB)])


def kernel(cam_id, r, t):
    mesh = plsc.VectorSubcoreMesh(core_axis_name="c", subcore_axis_name="s")
    rt_flat = jnp.concatenate([jnp.transpose(r), jnp.transpose(t)], axis=0)
    out_t = pl.kernel(
        _body,
        out_type=jax.ShapeDtypeStruct((4, NB, 4, 128), jnp.float32),
        mesh=mesh,
        compiler_params=pltpu.CompilerParams(
            skip_device_barrier=True, use_tc_tiling_on_sc=False),
        scratch_types=[
            pltpu.VMEM((BPW,), jnp.int32),
            pltpu.VMEM((6, BPW), jnp.float32),
            pltpu.VMEM((4, BPWB, 4, 128), jnp.float32),
            pltpu.SemaphoreType.DMA((6,)),
        ],
    )(cam_id.astype(jnp.int32), rt_flat)
    return jnp.transpose(out_t, (1, 3, 0, 2)).reshape(N_RAYS, 4, 4)
